# baseline (speedup 1.0000x reference)
import jax
import jax.numpy as jnp
from jax import lax
from jax.experimental import pallas as pl
from jax.experimental.pallas import tpu as pltpu
from jax.experimental.pallas import tpu_sc as plsc

N = 10000
E = 320000
D = 128
ALPHA = 0.2

NC = 2
NS = 16
NW = NC * NS
CH = 96
NCH = 105
EPT = NCH * CH
EPAD = NW * EPT
N2 = 10112
RPT = N2 // NS
ZR = 8

_f32 = jnp.float32
_i32 = jnp.int32
_bf16 = jnp.bfloat16


def _prep_body(x_ref, w_ref, a_ref, hb_ref, s_ref):
    h = jnp.dot(x_ref[...], w_ref[...], preferred_element_type=_f32)
    a2 = a_ref[...].reshape(2, D)
    s = lax.dot_general(h, a2, (((1,), (1,)), ((), ())),
                        preferred_element_type=_f32)
    s_ref[...] = s
    row = lax.broadcasted_iota(_i32, (D, D), 0)
    col = lax.broadcasted_iota(_i32, (D, D), 1)
    src_col = (col // 32) * 32 + (col % 2) * 16 + (col % 32) // 2
    perm = jnp.where(row == src_col, 1.0, 0.0).astype(_f32)
    hb_ref[...] = jnp.dot(h, perm, preferred_element_type=_f32).astype(_bf16)


def _tc_prep(x, W, a):
    blk = 1000
    grid = N // blk
    return pl.pallas_call(
        _prep_body,
        grid=(grid,),
        in_specs=[
            pl.BlockSpec((blk, D), lambda i: (i, 0)),
            pl.BlockSpec((D, D), lambda i: (0, 0)),
            pl.BlockSpec((1, 2 * D), lambda i: (0, 0)),
        ],
        out_specs=[
            pl.BlockSpec((blk, D), lambda i: (i, 0)),
            pl.BlockSpec((blk, 2), lambda i: (i, 0)),
        ],
        out_shape=[
            jax.ShapeDtypeStruct((N, D), _bf16),
            jax.ShapeDtypeStruct((N, 2), _f32),
        ],
    )(x, W, a)


def _sc_body(eidx_hbm, s1_hbm, s2_hbm, hb_hbm, u_out, dn_out,
             src_v, dst_b, sv1, sv2, pbuf, rows_bf, srows, zbuf, zdn,
             u_sh, dn_sh, gsem, ssem0, ssem1, dsem0, dsem1, sems, isem):
    cid = lax.axis_index("c")
    sid = lax.axis_index("s")
    wid = cid * NS + sid
    riota = lax.iota(_i32, 16)

    for i in range(ZR):
        for c in range(D // 16):
            zbuf[i, pl.ds(c * 16, 16)] = jnp.zeros((16,), _f32)
    for k in range(RPT // 16):
        zdn[pl.ds(k * 16, 16)] = jnp.zeros((16,), _f32)
    for k in range(RPT // ZR):
        pltpu.sync_copy(zbuf, u_sh.at[pl.ds(sid * RPT + k * ZR, ZR)])
    pltpu.sync_copy(zdn, dn_sh.at[pl.ds(sid * RPT, RPT)])

    pltpu.sync_copy(eidx_hbm.at[0, wid], src_v)
    plsc.subcore_barrier()

    pltpu.sync_copy(eidx_hbm.at[1, wid, 0], dst_b.at[0])
    pltpu.async_copy(s1_hbm.at[src_v.at[0]], sv1.at[0], sems)
    pltpu.async_copy(s2_hbm.at[dst_b.at[0]], sv2.at[0], sems)
    pltpu.async_copy(hb_hbm.at[dst_b.at[0]], rows_bf.at[0], gsem)
    pltpu.async_copy(eidx_hbm.at[1, wid, 1], dst_b.at[1], isem)

    def _chunk(j, _):
        b = lax.rem(j, 2)
        pltpu.make_async_copy(s1_hbm.at[src_v.at[j]], sv1.at[b], sems).wait()
        pltpu.make_async_copy(s2_hbm.at[dst_b.at[b]], sv2.at[b], sems).wait()

        @pl.when(j + 1 < NCH)
        def _():
            pltpu.make_async_copy(eidx_hbm.at[1, wid, j + 1],
                                  dst_b.at[1 - b], isem).wait()
            pltpu.async_copy(s1_hbm.at[src_v.at[j + 1]], sv1.at[1 - b], sems)
            pltpu.async_copy(s2_hbm.at[dst_b.at[1 - b]], sv2.at[1 - b], sems)
            pltpu.async_copy(hb_hbm.at[dst_b.at[1 - b]], rows_bf.at[1 - b],
                             gsem)

        @pl.when((j >= 2) & (b == 0))
        def _():
            pltpu.make_async_copy(pbuf.at[b], dn_sh.at[src_v.at[j - 2]],
                                  dsem0).wait()
            pltpu.make_async_copy(srows.at[b], u_sh.at[src_v.at[j - 2]],
                                  ssem0).wait()

        @pl.when((j >= 2) & (b == 1))
        def _():
            pltpu.make_async_copy(pbuf.at[b], dn_sh.at[src_v.at[j - 2]],
                                  dsem1).wait()
            pltpu.make_async_copy(srows.at[b], u_sh.at[src_v.at[j - 2]],
                                  ssem1).wait()

        base = wid * EPT + j * CH
        for m in range(CH // 16):
            sl = pl.ds(m * 16, 16)
            e = sv1[b, sl] + sv2[b, sl]
            e = jnp.where(e > 0, e, ALPHA * e)
            p = jnp.exp(e)
            pos = riota + (base + m * 16)
            pbuf[b, sl] = jnp.where(pos < E, p, 0.0)

        @pl.when(b == 0)
        def _():
            pltpu.async_copy(pbuf.at[b], dn_sh.at[src_v.at[j]], dsem0,
                             add=True)

        @pl.when(b == 1)
        def _():
            pltpu.async_copy(pbuf.at[b], dn_sh.at[src_v.at[j]], dsem1,
                             add=True)

        pltpu.make_async_copy(hb_hbm.at[dst_b.at[b]], rows_bf.at[b],
                              gsem).wait()

        @plsc.parallel_loop(0, CH, 1, unroll=8)
        def _scale(r):
            pb = plsc.load_gather(pbuf, [jnp.full((16,), b, _i32),
                                         jnp.full((16,), r, _i32)])
            for c in range(D // 32):
                v = rows_bf[b, r, pl.ds(c * 32, 32)]
                lo, hi = plsc.unpack(v, format=plsc.PackFormat.INTERLEAVED)
                srows[b, r, pl.ds(c * 32, 16)] = lo * pb
                srows[b, r, pl.ds(c * 32 + 16, 16)] = hi * pb

        @pl.when(b == 0)
        def _():
            pltpu.async_copy(srows.at[b], u_sh.at[src_v.at[j]], ssem0,
                             add=True)

        @pl.when(b == 1)
        def _():
            pltpu.async_copy(srows.at[b], u_sh.at[src_v.at[j]], ssem1,
                             add=True)

        @pl.when(j + 2 < NCH)
        def _():
            pltpu.async_copy(eidx_hbm.at[1, wid, j + 2], dst_b.at[b], isem)
        return 0

    lax.fori_loop(0, NCH, _chunk, 0)
    pltpu.make_async_copy(pbuf.at[0], dn_sh.at[src_v.at[0]], dsem0).wait()
    pltpu.make_async_copy(pbuf.at[1], dn_sh.at[src_v.at[0]], dsem1).wait()
    pltpu.make_async_copy(srows.at[0], u_sh.at[src_v.at[0]], ssem0).wait()
    pltpu.make_async_copy(srows.at[1], u_sh.at[src_v.at[0]], ssem1).wait()
    plsc.subcore_barrier()

    pltpu.sync_copy(u_sh.at[pl.ds(sid * RPT, RPT)],
                    u_out.at[cid, pl.ds(sid * RPT, RPT)])
    pltpu.sync_copy(dn_sh.at[pl.ds(sid * RPT, RPT)],
                    dn_out.at[cid, pl.ds(sid * RPT, RPT)])


def _sc_agg(eidx, s1, s2, hb):
    mesh = plsc.VectorSubcoreMesh(core_axis_name="c", subcore_axis_name="s")
    return pl.kernel(
        _sc_body,
        out_type=(jax.ShapeDtypeStruct((NC, N2, D), _f32),
                  jax.ShapeDtypeStruct((NC, N2), _f32)),
        mesh=mesh,
        compiler_params=pltpu.CompilerParams(
            needs_layout_passes=False, use_tc_tiling_on_sc=False),
        scratch_types=[
            pltpu.VMEM((NCH, CH), _i32),
            pltpu.VMEM((2, CH), _i32),
            pltpu.VMEM((2, CH), _f32),
            pltpu.VMEM((2, CH), _f32),
            pltpu.VMEM((2, CH), _f32),
            pltpu.VMEM((2, CH, D), _bf16),
            pltpu.VMEM((2, CH, D), _f32),
            pltpu.VMEM((ZR, D), _f32),
            pltpu.VMEM((RPT,), _f32),
            pltpu.VMEM_SHARED((N2, D), _f32),
            pltpu.VMEM_SHARED((N2,), _f32),
            pltpu.SemaphoreType.DMA,
            pltpu.SemaphoreType.DMA,
            pltpu.SemaphoreType.DMA,
            pltpu.SemaphoreType.DMA,
            pltpu.SemaphoreType.DMA,
            pltpu.SemaphoreType.DMA,
            pltpu.SemaphoreType.DMA,
        ],
    )(eidx, s1, s2, hb)


def _combine_body(u_ref, dn_ref, o_ref):
    num = u_ref[0] + u_ref[1]
    den = dn_ref[0] + dn_ref[1]
    den = jnp.where(den > 0, den, 1.0)
    r = num / den[:, None]
    o_ref[...] = jnp.where(r > 0, r, jnp.exp(jnp.minimum(r, 0.0)) - 1.0)


def _tc_combine(u, dn):
    blk = 128
    grid = N2 // blk
    return pl.pallas_call(
        _combine_body,
        grid=(grid,),
        in_specs=[
            pl.BlockSpec((NC, blk, D), lambda i: (0, i, 0)),
            pl.BlockSpec((NC, blk), lambda i: (0, i)),
        ],
        out_specs=pl.BlockSpec((blk, D), lambda i: (i, 0)),
        out_shape=jax.ShapeDtypeStruct((N, D), _f32),
    )(u, dn)


def kernel(x, edge_index, W, a):
    hb, s = _tc_prep(x, W, a)
    s1 = s[:, 0]
    s2 = s[:, 1]
    pad = jnp.zeros((2, EPAD - E), _i32)
    eidx = jnp.concatenate([edge_index, pad], axis=1).reshape(2, NW, NCH, CH)
    u, dn = _sc_agg(eidx, s1, s2, hb)
    return _tc_combine(u, dn)


# speedup vs baseline: 1.3454x; 1.3454x over previous
import jax
import jax.numpy as jnp
from jax import lax
from jax.experimental import pallas as pl
from jax.experimental.pallas import tpu as pltpu
from jax.experimental.pallas import tpu_sc as plsc

N = 10000
E = 320000
D = 128
ALPHA = 0.2

NC = 2
NS = 16
NW = NC * NS
EPT = E // NW
CH = 80
NCH = EPT // CH
N2 = 10240
RPT = N2 // NS
ZR = 16

_f32 = jnp.float32
_i32 = jnp.int32
_bf16 = jnp.bfloat16


def _prep_body(x_ref, w_ref, a_ref, hb_ref, s_ref):
    h = jnp.dot(x_ref[...], w_ref[...], preferred_element_type=_f32)
    a2 = a_ref[...].reshape(2, D)
    s = lax.dot_general(h, a2, (((1,), (1,)), ((), ())),
                        preferred_element_type=_f32)
    s_ref[...] = s
    row = lax.broadcasted_iota(_i32, (D, D), 0)
    col = lax.broadcasted_iota(_i32, (D, D), 1)
    src_col = (col // 32) * 32 + (col % 2) * 16 + (col % 32) // 2
    perm = jnp.where(row == src_col, 1.0, 0.0).astype(_f32)
    hb_ref[...] = jnp.dot(h, perm, preferred_element_type=_f32).astype(_bf16)


def _tc_prep(x, W, a):
    blk = 1000
    grid = N // blk
    return pl.pallas_call(
        _prep_body,
        grid=(grid,),
        in_specs=[
            pl.BlockSpec((blk, D), lambda i: (i, 0)),
            pl.BlockSpec((D, D), lambda i: (0, 0)),
            pl.BlockSpec((1, 2 * D), lambda i: (0, 0)),
        ],
        out_specs=[
            pl.BlockSpec((blk, D), lambda i: (i, 0)),
            pl.BlockSpec((blk, 2), lambda i: (i, 0)),
        ],
        out_shape=[
            jax.ShapeDtypeStruct((N, D), _bf16),
            jax.ShapeDtypeStruct((N, 2), _f32),
        ],
    )(x, W, a)


def _sc_body(eidx_hbm, s1_hbm, s2_hbm, hb_hbm, u_out, dn_out,
             src_v, dst_b, sv1, sv2, pbuf, rows_bf, srows, zbuf, zdn,
             u_sh, dn_sh, gsem, ssem0, ssem1, dsem0, dsem1, sems, isem):
    cid = lax.axis_index("c")
    sid = lax.axis_index("s")
    wid = cid * NS + sid

    for i in range(ZR):
        for c in range(D // 16):
            zbuf[i, pl.ds(c * 16, 16)] = jnp.zeros((16,), _f32)
    for k in range(RPT // 16):
        zdn[pl.ds(k * 16, 16)] = jnp.zeros((16,), _f32)
    for k in range(RPT // ZR):
        pltpu.sync_copy(zbuf, u_sh.at[pl.ds(sid * RPT + k * ZR, ZR)])
    pltpu.sync_copy(zdn, dn_sh.at[pl.ds(sid * RPT, RPT)])

    pltpu.sync_copy(eidx_hbm.at[0, wid], src_v)
    plsc.subcore_barrier()

    pltpu.sync_copy(eidx_hbm.at[1, wid, 0], dst_b.at[0])
    pltpu.async_copy(s1_hbm.at[src_v.at[0]], sv1.at[0], sems)
    pltpu.async_copy(s2_hbm.at[dst_b.at[0]], sv2.at[0], sems)
    pltpu.async_copy(hb_hbm.at[dst_b.at[0]], rows_bf.at[0], gsem)
    pltpu.async_copy(eidx_hbm.at[1, wid, 1], dst_b.at[1], isem)

    def _chunk(j, _):
        b = lax.rem(j, 2)
        pltpu.make_async_copy(s1_hbm.at[src_v.at[j]], sv1.at[b], sems).wait()
        pltpu.make_async_copy(s2_hbm.at[dst_b.at[b]], sv2.at[b], sems).wait()

        @pl.when(j + 1 < NCH)
        def _():
            pltpu.make_async_copy(eidx_hbm.at[1, wid, j + 1],
                                  dst_b.at[1 - b], isem).wait()
            pltpu.async_copy(s1_hbm.at[src_v.at[j + 1]], sv1.at[1 - b], sems)
            pltpu.async_copy(s2_hbm.at[dst_b.at[1 - b]], sv2.at[1 - b], sems)
            pltpu.async_copy(hb_hbm.at[dst_b.at[1 - b]], rows_bf.at[1 - b],
                             gsem)

        @pl.when((j >= 2) & (b == 0))
        def _():
            pltpu.make_async_copy(pbuf.at[b], dn_sh.at[src_v.at[j - 2]],
                                  dsem0).wait()
            pltpu.make_async_copy(srows.at[b], u_sh.at[src_v.at[j - 2]],
                                  ssem0).wait()

        @pl.when((j >= 2) & (b == 1))
        def _():
            pltpu.make_async_copy(pbuf.at[b], dn_sh.at[src_v.at[j - 2]],
                                  dsem1).wait()
            pltpu.make_async_copy(srows.at[b], u_sh.at[src_v.at[j - 2]],
                                  ssem1).wait()

        for m in range(CH // 16):
            sl = pl.ds(m * 16, 16)
            e = sv1[b, sl] + sv2[b, sl]
            e = jnp.where(e > 0, e, ALPHA * e)
            pbuf[b, sl] = jnp.exp(e)

        @pl.when(b == 0)
        def _():
            pltpu.async_copy(pbuf.at[b], dn_sh.at[src_v.at[j]], dsem0,
                             add=True)

        @pl.when(b == 1)
        def _():
            pltpu.async_copy(pbuf.at[b], dn_sh.at[src_v.at[j]], dsem1,
                             add=True)

        pltpu.make_async_copy(hb_hbm.at[dst_b.at[b]], rows_bf.at[b],
                              gsem).wait()

        @plsc.parallel_loop(0, CH, 1, unroll=8)
        def _scale(r):
            pb = plsc.load_gather(pbuf, [jnp.full((16,), b, _i32),
                                         jnp.full((16,), r, _i32)])
            for c in range(D // 32):
                v = rows_bf[b, r, pl.ds(c * 32, 32)]
                lo, hi = plsc.unpack(v, format=plsc.PackFormat.INTERLEAVED)
                srows[b, r, pl.ds(c * 32, 16)] = lo * pb
                srows[b, r, pl.ds(c * 32 + 16, 16)] = hi * pb

        @pl.when(b == 0)
        def _():
            pltpu.async_copy(srows.at[b], u_sh.at[src_v.at[j]], ssem0,
                             add=True)

        @pl.when(b == 1)
        def _():
            pltpu.async_copy(srows.at[b], u_sh.at[src_v.at[j]], ssem1,
                             add=True)

        @pl.when(j + 2 < NCH)
        def _():
            pltpu.async_copy(eidx_hbm.at[1, wid, j + 2], dst_b.at[b], isem)
        return 0

    lax.fori_loop(0, NCH, _chunk, 0)
    pltpu.make_async_copy(pbuf.at[0], dn_sh.at[src_v.at[0]], dsem0).wait()
    pltpu.make_async_copy(pbuf.at[1], dn_sh.at[src_v.at[0]], dsem1).wait()
    pltpu.make_async_copy(srows.at[0], u_sh.at[src_v.at[0]], ssem0).wait()
    pltpu.make_async_copy(srows.at[1], u_sh.at[src_v.at[0]], ssem1).wait()
    plsc.subcore_barrier()

    pltpu.sync_copy(u_sh.at[pl.ds(sid * RPT, RPT)],
                    u_out.at[cid, pl.ds(sid * RPT, RPT)])
    pltpu.sync_copy(dn_sh.at[pl.ds(sid * RPT, RPT)],
                    dn_out.at[cid, pl.ds(sid * RPT, RPT)])


def _sc_agg(eidx, s1, s2, hb):
    mesh = plsc.VectorSubcoreMesh(core_axis_name="c", subcore_axis_name="s")
    return pl.kernel(
        _sc_body,
        out_type=(jax.ShapeDtypeStruct((NC, N2, D), _f32),
                  jax.ShapeDtypeStruct((NC, N2), _f32)),
        mesh=mesh,
        compiler_params=pltpu.CompilerParams(
            needs_layout_passes=False, use_tc_tiling_on_sc=False),
        scratch_types=[
            pltpu.VMEM((NCH, CH), _i32),
            pltpu.VMEM((2, CH), _i32),
            pltpu.VMEM((2, CH), _f32),
            pltpu.VMEM((2, CH), _f32),
            pltpu.VMEM((2, CH), _f32),
            pltpu.VMEM((2, CH, D), _bf16),
            pltpu.VMEM((2, CH, D), _f32),
            pltpu.VMEM((ZR, D), _f32),
            pltpu.VMEM((RPT,), _f32),
            pltpu.VMEM_SHARED((N2, D), _f32),
            pltpu.VMEM_SHARED((N2,), _f32),
            pltpu.SemaphoreType.DMA,
            pltpu.SemaphoreType.DMA,
            pltpu.SemaphoreType.DMA,
            pltpu.SemaphoreType.DMA,
            pltpu.SemaphoreType.DMA,
            pltpu.SemaphoreType.DMA,
            pltpu.SemaphoreType.DMA,
        ],
    )(eidx, s1, s2, hb)


def _combine_body(u_ref, dn_ref, o_ref):
    num = u_ref[0] + u_ref[1]
    den = dn_ref[0] + dn_ref[1]
    den = jnp.where(den > 0, den, 1.0)
    r = num / den[:, None]
    o_ref[...] = jnp.where(r > 0, r, jnp.exp(jnp.minimum(r, 0.0)) - 1.0)


def _tc_combine(u, dn):
    blk = 1024
    grid = N2 // blk
    return pl.pallas_call(
        _combine_body,
        grid=(grid,),
        in_specs=[
            pl.BlockSpec((NC, blk, D), lambda i: (0, i, 0)),
            pl.BlockSpec((NC, blk), lambda i: (0, i)),
        ],
        out_specs=pl.BlockSpec((blk, D), lambda i: (i, 0)),
        out_shape=jax.ShapeDtypeStruct((N, D), _f32),
    )(u, dn)


def kernel(x, edge_index, W, a):
    hb, s = _tc_prep(x, W, a)
    s1 = s[:, 0]
    s2 = s[:, 1]
    eidx = edge_index.reshape(2, NW, NCH, CH)
    u, dn = _sc_agg(eidx, s1, s2, hb)
    return _tc_combine(u, dn)
